# baseline (speedup 1.0000x reference)
import functools

import jax
import jax.numpy as jnp
from jax import lax
from jax.experimental import pallas as pl
from jax.experimental.pallas import tpu as pltpu
from jax.experimental.pallas import tpu_sc as plsc

D = 1024
NH = 16
HD = 64
BATCH = 4
S = 2048
K_SEL = max(1, int(S * 0.1))
K_PAD = 256
QB = 256
NQ = S // QB
NEG = -1e30
L = 16
NSLICE = S // L


def _gate_body(x_ref, g1_ref, g1b_ref, g2_ref, g2b_ref, imp_ref):
    f32 = jnp.float32
    xb = x_ref[0]
    hgt = lax.dot_general(g1_ref[...], xb, (((1,), (1,)), ((), ())),
                          preferred_element_type=f32)
    hgt = jnp.maximum(hgt + g1b_ref[...], 0.0)
    logit = lax.dot_general(g2_ref[...], hgt, (((1,), (0,)), ((), ())),
                            preferred_element_type=f32)
    logit = logit + g2b_ref[0, 0]
    imp = 1.0 / (1.0 + jnp.exp(-logit))
    imp_ref[0] = lax.bitcast_convert_type(imp, jnp.int32)


def _gate(x, g1_w, g1b, g2_w, g2b):
    return pl.pallas_call(
        _gate_body,
        grid=(BATCH,),
        in_specs=[
            pl.BlockSpec((1, S, D), lambda b: (b, 0, 0)),
            pl.BlockSpec((D // 4, D), lambda b: (0, 0)),
            pl.BlockSpec((D // 4, 1), lambda b: (0, 0)),
            pl.BlockSpec((1, D // 4), lambda b: (0, 0)),
            pl.BlockSpec((1, 1), lambda b: (0, 0)),
        ],
        out_specs=pl.BlockSpec((1, 1, S), lambda b: (b, 0, 0)),
        out_shape=jax.ShapeDtypeStruct((BATCH, 1, S), jnp.int32),
    )(x, g1_w, g1b, g2_w, g2b)


def _sc_select(impbits):
    mesh = plsc.VectorSubcoreMesh(core_axis_name="c", subcore_axis_name="s",
                                  num_cores=2, num_subcores=16)

    @functools.partial(
        pl.kernel, mesh=mesh,
        out_type=jax.ShapeDtypeStruct((BATCH, 2, L), jnp.int32),
        scratch_types=[
            pltpu.VMEM((S,), jnp.int32),
            pltpu.VMEM((2 * L,), jnp.int32),
            pltpu.VMEM((2, L), jnp.int32),
        ],
    )
    def sel_kernel(imp_hbm, tr_hbm, imp_v, red_v, tr_v):
        wid = lax.axis_index("s") * 2 + lax.axis_index("c")

        @pl.when(wid < BATCH)
        def _():
            pltpu.sync_copy(imp_hbm.at[wid], imp_v)
            zeros = jnp.zeros((L,), jnp.int32)
            ones = jnp.ones((L,), jnp.int32)
            ksel = jnp.full((L,), K_SEL, jnp.int32)

            def lane_total(acc):
                red_v[pl.ds(0, L)] = acc
                red_v[pl.ds(L, L)] = acc
                for sft in (8, 4, 2, 1):
                    c = red_v[pl.ds(0, L)] + red_v[pl.ds(sft, L)]
                    red_v[pl.ds(0, L)] = c
                    red_v[pl.ds(L, L)] = c
                return red_v[pl.ds(0, L)]

            def count_ge(mid):
                def cbody(j, acc):
                    v = imp_v[pl.ds(j * L, L)]
                    return acc + jnp.where(v >= mid, ones, zeros)
                acc = lax.fori_loop(0, NSLICE, cbody, zeros)
                return lane_total(acc)

            def bs(_, carry):
                lo, hi = carry
                mid = (lo + hi) >> 1
                take = count_ge(mid) >= ksel
                return (jnp.where(take, mid, lo), jnp.where(take, hi, mid))

            t, _unused = lax.fori_loop(
                0, 31, bs, (zeros, jnp.full((L,), 0x3F800001, jnp.int32)))

            def cgt(j, acc):
                v = imp_v[pl.ds(j * L, L)]
                return acc + jnp.where(v > t, ones, zeros)
            n_gt = lane_total(lax.fori_loop(0, NSLICE, cgt, zeros))

            tr_v[0] = t
            tr_v[1] = ksel - n_gt
            pltpu.sync_copy(tr_v, tr_hbm.at[wid])

    return sel_kernel(impbits)


def _cumsum_lanes(a, n):
    sh = 1
    while sh < n:
        shifted = jnp.concatenate(
            [jnp.zeros((1, sh), jnp.int32), a[:, : n - sh]], axis=1)
        a = a + shifted
        sh *= 2
    return a


def _attn_body(x_ref, xq_ref, bits_ref, tr_ref, wqkv_ref, bqkv_ref,
               wo_ref, bo_ref, out_ref, ksel_ref, vsel_ref):
    qi = pl.program_id(1)
    f32 = jnp.float32

    @pl.when(qi == 0)
    def _phase_a():
        xb = x_ref[0]
        bits = bits_ref[0]
        t = tr_ref[0, 0, 0]
        r = tr_ref[0, 1, 0]
        gt = bits > t
        tie = bits == t
        tie_rank = _cumsum_lanes(tie.astype(jnp.int32), S)
        sel = gt | (tie & (tie_rank <= r))
        selr = _cumsum_lanes(sel.astype(jnp.int32), S) - 1
        selr = jnp.where(sel, selr, -1)

        rows = lax.broadcasted_iota(jnp.int32, (K_PAD, S), 0)
        p = (rows == selr).astype(f32)
        x_sel = lax.dot_general(p, xb, (((1,), (0,)), ((), ())),
                                preferred_element_type=f32)
        wk = wqkv_ref[D:2 * D, :]
        wv = wqkv_ref[2 * D:3 * D, :]
        ksel_ref[...] = lax.dot_general(
            x_sel, wk, (((1,), (1,)), ((), ())),
            preferred_element_type=f32) + bqkv_ref[1:2, :]
        vsel_ref[...] = lax.dot_general(
            x_sel, wv, (((1,), (1,)), ((), ())),
            preferred_element_type=f32) + bqkv_ref[2:3, :]

    xq = xq_ref[0]
    wq = wqkv_ref[0:D, :]
    q = lax.dot_general(xq, wq, (((1,), (1,)), ((), ())),
                        preferred_element_type=f32) + bqkv_ref[0:1, :]
    ksel = ksel_ref[...]
    vsel = vsel_ref[...]
    col = lax.broadcasted_iota(jnp.int32, (QB, K_PAD), 1)
    pad_bias = jnp.where(col < K_SEL, 0.0, NEG)

    outs = []
    for h in range(NH):
        sl = slice(h * HD, (h + 1) * HD)
        s = lax.dot_general(q[:, sl], ksel[:, sl], (((1,), (1,)), ((), ())),
                            preferred_element_type=f32)
        s = s * (1.0 / (HD ** 0.5)) + pad_bias
        m = jnp.max(s, axis=1, keepdims=True)
        e = jnp.exp(s - m)
        l = jnp.sum(e, axis=1, keepdims=True)
        oh = lax.dot_general(e, vsel[:, sl], (((1,), (0,)), ((), ())),
                             preferred_element_type=f32) / l
        outs.append(oh)
    o = jnp.concatenate(outs, axis=1)
    res = lax.dot_general(o, wo_ref[...], (((1,), (1,)), ((), ())),
                          preferred_element_type=f32) + bo_ref[...]
    out_ref[0] = res


def _attn(x, bits3, tr, in_proj_w, bqkv, out_proj_w, bo):
    return pl.pallas_call(
        _attn_body,
        grid=(BATCH, NQ),
        in_specs=[
            pl.BlockSpec((1, S, D), lambda b, q: (b, 0, 0)),
            pl.BlockSpec((1, QB, D), lambda b, q: (b, q, 0)),
            pl.BlockSpec((1, 1, S), lambda b, q: (b, 0, 0)),
            pl.BlockSpec((1, 2, L), lambda b, q: (b, 0, 0)),
            pl.BlockSpec((3 * D, D), lambda b, q: (0, 0)),
            pl.BlockSpec((3, D), lambda b, q: (0, 0)),
            pl.BlockSpec((D, D), lambda b, q: (0, 0)),
            pl.BlockSpec((1, D), lambda b, q: (0, 0)),
        ],
        out_specs=pl.BlockSpec((1, QB, D), lambda b, q: (b, q, 0)),
        out_shape=jax.ShapeDtypeStruct((BATCH, S, D), jnp.float32),
        scratch_shapes=[
            pltpu.VMEM((K_PAD, D), jnp.float32),
            pltpu.VMEM((K_PAD, D), jnp.float32),
        ],
        compiler_params=pltpu.CompilerParams(
            dimension_semantics=("arbitrary", "arbitrary")),
    )(x, x, bits3, tr, in_proj_w, bqkv, out_proj_w, bo)


@jax.jit
def kernel(x, in_proj_w, in_proj_b, out_proj_w, out_proj_b,
           g1_w, g1_b, g2_w, g2_b):
    bqkv = in_proj_b.reshape(3, D)
    bo = out_proj_b.reshape(1, D)
    g1b = g1_b.reshape(D // 4, 1)
    g2b = g2_b.reshape(1, 1)

    bits = _gate(x, g1_w, g1b, g2_w, g2b)
    tr = _sc_select(bits.reshape(BATCH, S))
    return _attn(x, bits, tr, in_proj_w, bqkv, out_proj_w, bo)
